# baseline (speedup 1.0000x reference)
import functools
import jax
import jax.numpy as jnp
from jax import lax
from jax.experimental import pallas as pl
from jax.experimental.pallas import tpu as pltpu
from jax.experimental.pallas import tpu_sc as plsc

N_NODES = 10000
NC = 2
NS = 16
NW = NC * NS
L = 16

RPT = (N_NODES + NW - 1) // NW
ACC_ROWS = 314
TRASH = ACC_ROWS - 1
NPAD = N_NODES + 48
ADT = 328
SUB = 4096
SLOT = SUB + 2 * L
G = 16
NEG_SLOPE = 0.2


def _tc_matmul_fn(x_ref, w_ref, a_ref, h_ref, aux_ref, mx_ref):
    h = jnp.dot(x_ref[...], w_ref[...], preferred_element_type=jnp.float32)
    h_ref[...] = h
    aux = jnp.dot(h, a_ref[...], preferred_element_type=jnp.float32)
    aux_ref[...] = aux
    mblk = jnp.max(aux, axis=0, keepdims=True)

    @pl.when(pl.program_id(0) == 0)
    def _():
        mx_ref[...] = mblk

    @pl.when(pl.program_id(0) > 0)
    def _():
        mx_ref[...] = jnp.maximum(mx_ref[...], mblk)


def _tc_matmul(x, w, attmat):
    n, din = x.shape
    dout = w.shape[1]
    blk = 1000
    grid = (n // blk,)
    return pl.pallas_call(
        _tc_matmul_fn,
        grid=grid,
        in_specs=[
            pl.BlockSpec((blk, din), lambda i: (i, 0)),
            pl.BlockSpec((din, dout), lambda i: (0, 0)),
            pl.BlockSpec((dout, 128), lambda i: (0, 0)),
        ],
        out_specs=[
            pl.BlockSpec((blk, dout), lambda i: (i, 0)),
            pl.BlockSpec((blk, 128), lambda i: (i, 0)),
            pl.BlockSpec((1, 128), lambda i: (0, 0)),
        ],
        out_shape=[
            jax.ShapeDtypeStruct((n, dout), jnp.float32),
            jax.ShapeDtypeStruct((n, 128), jnp.float32),
            jax.ShapeDtypeStruct((1, 128), jnp.float32),
        ],
    )(x, w, attmat)


def _mesh():
    return plsc.VectorSubcoreMesh(
        core_axis_name="c", subcore_axis_name="s", num_cores=NC,
        num_subcores=NS)


def _make_sc_partition(tot: int):
    NSUB = tot // SUB
    NV = SUB // L

    @functools.partial(
        pl.kernel,
        out_type=[
            jax.ShapeDtypeStruct((NW, NSUB, SLOT), jnp.int32),
            jax.ShapeDtypeStruct((NW, NSUB, SLOT), jnp.int32),
        ],
        mesh=_mesh(),
        compiler_params=pltpu.CompilerParams(needs_layout_passes=False),
        scratch_types=[
            pltpu.VMEM((SUB,), jnp.int32),
            pltpu.VMEM((SUB,), jnp.int32),
            pltpu.VMEM((SLOT,), jnp.int32),
            pltpu.VMEM((SLOT,), jnp.int32),
        ],
    )
    def sc_part(src_h, dst_h, esrc_h, edl_h, slb, dstv, slist, dlist):
        c = lax.axis_index("c")
        s = lax.axis_index("s")
        wid = s * NC + c
        lo = wid * RPT
        hi = jnp.minimum(lo + RPT, N_NODES)

        def sub_body(sub, _):
            base = sub * SUB
            pltpu.sync_copy(src_h.at[pl.ds(base, SUB)], slb)
            pltpu.sync_copy(dst_h.at[pl.ds(base, SUB)], dstv)

            def vec_body(v, off):
                s16 = slb[pl.ds(v * L, L)]
                d16 = dstv[pl.ds(v * L, L)]
                keep = (d16 >= lo) & (d16 < hi)
                dl = d16 - lo
                plsc.store_compressed(slist.at[pl.ds(off, L)], s16, mask=keep)
                plsc.store_compressed(dlist.at[pl.ds(off, L)], dl, mask=keep)
                cnt = plsc.all_reduce_population_count(keep)
                return off + cnt[0]
            cnt = lax.fori_loop(0, NV, vec_body, jnp.int32(0))

            slist[pl.ds(cnt, L)] = jnp.zeros((L,), jnp.int32)
            dlist[pl.ds(cnt, L)] = jnp.full((L,), TRASH, jnp.int32)
            ng = (cnt + (L - 1)) // L
            dlist[pl.ds(SLOT - L, L)] = jnp.full((L,), ng, jnp.int32)

            pltpu.sync_copy(slist, esrc_h.at[wid, sub])
            pltpu.sync_copy(dlist, edl_h.at[wid, sub])
            return 0
        lax.fori_loop(0, NSUB, sub_body, 0)

    return sc_part


def _make_sc_gat(d: int, tot: int, relu: bool):
    NSUB = tot // SUB

    @functools.partial(
        pl.kernel,
        out_type=jax.ShapeDtypeStruct((N_NODES, d), jnp.float32),
        mesh=_mesh(),
        compiler_params=pltpu.CompilerParams(needs_layout_passes=False),
        scratch_types=[
            pltpu.VMEM((ACC_ROWS, d), jnp.float32),
            pltpu.VMEM((ACC_ROWS,), jnp.float32),
            pltpu.VMEM((NPAD,), jnp.float32),
            pltpu.VMEM((ADT,), jnp.float32),
            pltpu.VMEM((128,), jnp.float32),
            pltpu.VMEM((d,), jnp.float32),
            pltpu.VMEM((SLOT,), jnp.int32),
            pltpu.VMEM((SLOT,), jnp.int32),
            pltpu.VMEM((G, d), jnp.float32),
            pltpu.SemaphoreType.DMA,
        ],
    )
    def sc_gat(esrc_h, edl_h, as_h, ad_h, mx_h, feat_h, b_h,
               out_h, acc, den, as_t, ad_t, mx_t, b_t,
               slist, dlist, rows, sem):
        c = lax.axis_index("c")
        s = lax.axis_index("s")
        wid = s * NC + c
        lo = wid * RPT
        hi = jnp.minimum(lo + RPT, N_NODES)
        albase = (lo // 8) * 8
        delta = lo - albase

        zvec = jnp.zeros((L,), jnp.float32)

        def zbody(r, _):
            for k in range(d // L):
                acc[r, pl.ds(k * L, L)] = zvec
            return 0
        lax.fori_loop(0, ACC_ROWS, zbody, 0)

        def zdbody(r, _):
            den[pl.ds(r * L, L)] = zvec
            return 0
        lax.fori_loop(0, ACC_ROWS // L, zdbody, 0)

        pltpu.sync_copy(as_h, as_t)
        pltpu.sync_copy(ad_h.at[pl.ds(albase, ADT)], ad_t)
        pltpu.sync_copy(mx_h, mx_t)
        pltpu.sync_copy(b_h, b_t)

        mrow = mx_t[pl.ds(0, L)]
        msum = mrow[0] + mrow[1]
        mv = jnp.full((L,), msum, jnp.float32)
        mv = jnp.where(mv >= 0.0, mv, mv * NEG_SLOPE)

        def sub_body(sub, _):
            pltpu.sync_copy(esrc_h.at[wid, sub], slist)
            pltpu.sync_copy(edl_h.at[wid, sub], dlist)
            tail = dlist[pl.ds(SLOT - L, L)]
            ng = tail[0]

            rowiota = lax.iota(jnp.int32, L)

            def grp_body(g, _):
                cp = pltpu.async_copy(
                    feat_h.at[slist.at[pl.ds(g * G, G)]], rows, sem)
                s16 = slist[pl.ds(g * G, G)]
                dl16 = dlist[pl.ds(g * G, G)]
                av = plsc.load_gather(as_t, [s16])
                dv = plsc.load_gather(ad_t, [dl16 + delta])
                e = av + dv
                e = jnp.where(e >= 0.0, e, e * NEG_SLOPE)
                w = jnp.exp(e - mv)
                plsc.addupdate_scatter(den, [dl16], w)
                cp.wait()

                def col_body(kb, _):
                    cv = jnp.full((L,), kb * L, jnp.int32)
                    for _t in range(L):
                        vals = plsc.load_gather(rows, [rowiota, cv])
                        plsc.addupdate_scatter(acc, [dl16, cv], vals * w)
                        cv = cv + 1
                    return 0
                lax.fori_loop(0, d // L, col_body, 0)
                return 0
            lax.fori_loop(0, ng, grp_body, 0)
            return 0
        lax.fori_loop(0, NSUB, sub_body, 0)

        def norm_body(rg, _):
            dch = den[pl.ds(rg * L, L)]
            for j in range(L):
                r = rg * L + j

                @pl.when(lo + r < hi)
                def _():
                    dj = jnp.maximum(dch[j], 1e-30)
                    for k in range(d // L):
                        sl = pl.ds(k * L, L)
                        val = acc[r, sl] / dj + b_t[sl]
                        if relu:
                            val = jnp.maximum(val, 0.0)
                        rows[0, sl] = val
                    pltpu.sync_copy(rows.at[pl.ds(0, 1)],
                                    out_h.at[pl.ds(lo + r, 1)])
            return 0
        lax.fori_loop(0, (RPT + L - 1) // L, norm_body, 0)

    return sc_gat


def kernel(x, edge_index, W1, att_src1, att_dst1, b1,
           W2, att_src2, att_dst2, b2):
    n = N_NODES
    e_in = edge_index.shape[1]
    n_edges = e_in + n
    tot = ((n_edges + SUB - 1) // SUB) * SUB

    loop = jnp.arange(n, dtype=jnp.int32)
    src = jnp.concatenate([edge_index[0], loop])
    dst = jnp.concatenate([edge_index[1], loop])
    pad = tot - n_edges
    src_p = jnp.pad(src, (0, pad))
    dst_p = jnp.pad(dst, (0, pad), constant_values=n)

    hid = W1.shape[1]
    out_d = W2.shape[1]
    att1 = jnp.zeros((hid, 128), jnp.float32)
    att1 = att1.at[:, 0].set(att_src1).at[:, 1].set(att_dst1)
    att2 = jnp.zeros((out_d, 128), jnp.float32)
    att2 = att2.at[:, 0].set(att_src2).at[:, 1].set(att_dst2)

    esrc, edl = _make_sc_partition(tot)(src_p, dst_p)

    h1, aux1, mx1 = _tc_matmul(x, W1, att1)
    as1 = jnp.pad(aux1[:, 0], (0, NPAD - n))
    ad1 = jnp.pad(aux1[:, 1], (0, NPAD - n))
    sc1 = _make_sc_gat(hid, tot, relu=True)
    h = sc1(esrc, edl, as1, ad1, mx1.reshape(128), h1, b1)

    h2, aux2, mx2 = _tc_matmul(h, W2, att2)
    as2 = jnp.pad(aux2[:, 0], (0, NPAD - n))
    ad2 = jnp.pad(aux2[:, 1], (0, NPAD - n))
    sc2 = _make_sc_gat(out_d, tot, relu=False)
    x2 = sc2(esrc, edl, as2, ad2, mx2.reshape(128), h2, b2)

    return x2, h


# speedup vs baseline: 2.3777x; 2.3777x over previous
import functools
import jax
import jax.numpy as jnp
from jax import lax
from jax.experimental import pallas as pl
from jax.experimental.pallas import tpu as pltpu
from jax.experimental.pallas import tpu_sc as plsc

N_NODES = 10000
NC = 2
NS = 16
NW = NC * NS
L = 16

RPT = (N_NODES + NW - 1) // NW
ACC_ROWS = 314
TRASH = ACC_ROWS - 1
NPAD = N_NODES + 48
ADT = 328
SUB = 4096
SLOT = SUB + 2 * L
G = 16
NEG_SLOPE = 0.2


def _tc_matmul_fn(x_ref, w_ref, a_ref, h_ref, aux_ref, mx_ref):
    h = jnp.dot(x_ref[...], w_ref[...], preferred_element_type=jnp.float32)
    h_ref[...] = h
    aux = jnp.dot(h, a_ref[...], preferred_element_type=jnp.float32)
    aux_ref[...] = aux
    mblk = jnp.max(aux, axis=0, keepdims=True)

    @pl.when(pl.program_id(0) == 0)
    def _():
        mx_ref[...] = mblk

    @pl.when(pl.program_id(0) > 0)
    def _():
        mx_ref[...] = jnp.maximum(mx_ref[...], mblk)


def _tc_matmul(x, w, attmat):
    n, din = x.shape
    dout = w.shape[1]
    blk = 1000
    grid = (n // blk,)
    return pl.pallas_call(
        _tc_matmul_fn,
        grid=grid,
        in_specs=[
            pl.BlockSpec((blk, din), lambda i: (i, 0)),
            pl.BlockSpec((din, dout), lambda i: (0, 0)),
            pl.BlockSpec((dout, 128), lambda i: (0, 0)),
        ],
        out_specs=[
            pl.BlockSpec((blk, dout), lambda i: (i, 0)),
            pl.BlockSpec((blk, 128), lambda i: (i, 0)),
            pl.BlockSpec((1, 128), lambda i: (0, 0)),
        ],
        out_shape=[
            jax.ShapeDtypeStruct((n, dout), jnp.float32),
            jax.ShapeDtypeStruct((n, 128), jnp.float32),
            jax.ShapeDtypeStruct((1, 128), jnp.float32),
        ],
    )(x, w, attmat)


def _mesh():
    return plsc.VectorSubcoreMesh(
        core_axis_name="c", subcore_axis_name="s", num_cores=NC,
        num_subcores=NS)


def _make_sc_partition(tot: int):
    NSUB = tot // SUB
    NV = SUB // L

    @functools.partial(
        pl.kernel,
        out_type=[
            jax.ShapeDtypeStruct((NW, NSUB, SLOT), jnp.int32),
            jax.ShapeDtypeStruct((NW, NSUB, SLOT), jnp.int32),
        ],
        mesh=_mesh(),
        compiler_params=pltpu.CompilerParams(needs_layout_passes=False),
        scratch_types=[
            pltpu.VMEM((SUB,), jnp.int32),
            pltpu.VMEM((SUB,), jnp.int32),
            pltpu.VMEM((SLOT,), jnp.int32),
            pltpu.VMEM((SLOT,), jnp.int32),
        ],
    )
    def sc_part(src_h, dst_h, esrc_h, edl_h, slb, dstv, slist, dlist):
        c = lax.axis_index("c")
        s = lax.axis_index("s")
        wid = s * NC + c
        lo = wid * RPT
        hi = jnp.minimum(lo + RPT, N_NODES)

        def sub_body(sub, _):
            base = sub * SUB
            pltpu.sync_copy(src_h.at[pl.ds(base, SUB)], slb)
            pltpu.sync_copy(dst_h.at[pl.ds(base, SUB)], dstv)

            def vec_body(v, off):
                s16 = slb[pl.ds(v * L, L)]
                d16 = dstv[pl.ds(v * L, L)]
                keep = (d16 >= lo) & (d16 < hi)
                dl = d16 - lo
                plsc.store_compressed(slist.at[pl.ds(off, L)], s16, mask=keep)
                plsc.store_compressed(dlist.at[pl.ds(off, L)], dl, mask=keep)
                cnt = plsc.all_reduce_population_count(keep)
                return off + cnt[0]
            cnt = lax.fori_loop(0, NV, vec_body, jnp.int32(0))

            slist[pl.ds(cnt, L)] = jnp.zeros((L,), jnp.int32)
            dlist[pl.ds(cnt, L)] = jnp.full((L,), TRASH, jnp.int32)
            ng = (cnt + (L - 1)) // L
            dlist[pl.ds(SLOT - L, L)] = jnp.full((L,), ng, jnp.int32)

            pltpu.sync_copy(slist, esrc_h.at[wid, sub])
            pltpu.sync_copy(dlist, edl_h.at[wid, sub])
            return 0
        lax.fori_loop(0, NSUB, sub_body, 0)

    return sc_part


def _make_sc_gat(d: int, tot: int, relu: bool):
    NSUB = tot // SUB

    @functools.partial(
        pl.kernel,
        out_type=jax.ShapeDtypeStruct((N_NODES, d), jnp.float32),
        mesh=_mesh(),
        compiler_params=pltpu.CompilerParams(needs_layout_passes=False),
        scratch_types=[
            pltpu.VMEM((ACC_ROWS, d), jnp.float32),
            pltpu.VMEM((ACC_ROWS * L,), jnp.float32),
            pltpu.VMEM((NPAD,), jnp.float32),
            pltpu.VMEM((ADT,), jnp.float32),
            pltpu.VMEM((128,), jnp.float32),
            pltpu.VMEM((d,), jnp.float32),
            pltpu.VMEM((SLOT,), jnp.int32),
            pltpu.VMEM((SLOT,), jnp.int32),
            pltpu.VMEM((G, d), jnp.float32),
            pltpu.SemaphoreType.DMA,
        ],
    )
    def sc_gat(esrc_h, edl_h, as_h, ad_h, mx_h, feat_h, b_h,
               out_h, acc, den, as_t, ad_t, mx_t, b_t,
               slist, dlist, rows, sem):
        c = lax.axis_index("c")
        s = lax.axis_index("s")
        wid = s * NC + c
        lo = wid * RPT
        hi = jnp.minimum(lo + RPT, N_NODES)
        albase = (lo // 8) * 8
        delta = lo - albase

        zvec = jnp.zeros((L,), jnp.float32)

        def zbody(r, _):
            for k in range(d // L):
                acc[r, pl.ds(k * L, L)] = zvec
            den[pl.ds(r * L, L)] = zvec
            return 0
        lax.fori_loop(0, ACC_ROWS, zbody, 0)

        pltpu.sync_copy(as_h, as_t)
        pltpu.sync_copy(ad_h.at[pl.ds(albase, ADT)], ad_t)
        pltpu.sync_copy(mx_h, mx_t)
        pltpu.sync_copy(b_h, b_t)

        mrow = mx_t[pl.ds(0, L)]
        msum = mrow[0] + mrow[1]
        mv = jnp.full((L,), msum, jnp.float32)
        mv = jnp.where(mv >= 0.0, mv, mv * NEG_SLOPE)

        def sub_body(sub, _):
            pltpu.sync_copy(esrc_h.at[wid, sub], slist)
            pltpu.sync_copy(edl_h.at[wid, sub], dlist)
            tail = dlist[pl.ds(SLOT - L, L)]
            ng = tail[0]

            def grp_body(g, _):
                cp = pltpu.async_copy(
                    feat_h.at[slist.at[pl.ds(g * G, G)]], rows, sem)
                s16 = slist[pl.ds(g * G, G)]
                dl16 = dlist[pl.ds(g * G, G)]
                av = plsc.load_gather(as_t, [s16])
                dv = plsc.load_gather(ad_t, [dl16 + delta])
                e = av + dv
                e = jnp.where(e >= 0.0, e, e * NEG_SLOPE)
                w = jnp.exp(e - mv)
                cp.wait()
                for j in range(G):
                    dlj = dl16[j]
                    wj = w[j]
                    for k in range(d // L):
                        sl = pl.ds(k * L, L)
                        plsc.addupdate(acc.at[dlj, sl], rows[j, sl] * wj)
                    plsc.addupdate(den.at[pl.ds(dlj * L, L)],
                                   jnp.full((L,), wj, jnp.float32))
                return 0
            lax.fori_loop(0, ng, grp_body, 0)
            return 0
        lax.fori_loop(0, NSUB, sub_body, 0)

        def norm_body(r, _):
            @pl.when(lo + r < hi)
            def _():
                dch = den[pl.ds(r * L, L)]
                dj = jnp.maximum(dch[0], 1e-30)
                for k in range(d // L):
                    sl = pl.ds(k * L, L)
                    val = acc[r, sl] / dj + b_t[sl]
                    if relu:
                        val = jnp.maximum(val, 0.0)
                    rows[0, sl] = val
                pltpu.sync_copy(rows.at[pl.ds(0, 1)],
                                out_h.at[pl.ds(lo + r, 1)])
            return 0
        lax.fori_loop(0, RPT, norm_body, 0)

    return sc_gat


def kernel(x, edge_index, W1, att_src1, att_dst1, b1,
           W2, att_src2, att_dst2, b2):
    n = N_NODES
    e_in = edge_index.shape[1]
    n_edges = e_in + n
    tot = ((n_edges + SUB - 1) // SUB) * SUB

    loop = jnp.arange(n, dtype=jnp.int32)
    src = jnp.concatenate([edge_index[0], loop])
    dst = jnp.concatenate([edge_index[1], loop])
    pad = tot - n_edges
    src_p = jnp.pad(src, (0, pad))
    dst_p = jnp.pad(dst, (0, pad), constant_values=n)

    hid = W1.shape[1]
    out_d = W2.shape[1]
    att1 = jnp.zeros((hid, 128), jnp.float32)
    att1 = att1.at[:, 0].set(att_src1).at[:, 1].set(att_dst1)
    att2 = jnp.zeros((out_d, 128), jnp.float32)
    att2 = att2.at[:, 0].set(att_src2).at[:, 1].set(att_dst2)

    esrc, edl = _make_sc_partition(tot)(src_p, dst_p)

    h1, aux1, mx1 = _tc_matmul(x, W1, att1)
    as1 = jnp.pad(aux1[:, 0], (0, NPAD - n))
    ad1 = jnp.pad(aux1[:, 1], (0, NPAD - n))
    sc1 = _make_sc_gat(hid, tot, relu=True)
    h = sc1(esrc, edl, as1, ad1, mx1.reshape(128), h1, b1)

    h2, aux2, mx2 = _tc_matmul(h, W2, att2)
    as2 = jnp.pad(aux2[:, 0], (0, NPAD - n))
    ad2 = jnp.pad(aux2[:, 1], (0, NPAD - n))
    sc2 = _make_sc_gat(out_d, tot, relu=False)
    x2 = sc2(esrc, edl, as2, ad2, mx2.reshape(128), h2, b2)

    return x2, h
